# baseline (speedup 1.0000x reference)
import jax
import jax.numpy as jnp
from jax import lax
from jax.experimental import pallas as pl
from jax.experimental.pallas import tpu as pltpu
from jax.experimental.pallas import tpu_sc as plsc

NN = 50000
EE = 800000
DD = 64
GG = 1024
CC = 10

BN = 256
NB = 196
NPAD = NB * BN

NSC = 2
NTL = 16
HALF = NN // NSC
ACC = 25600
ZPT = ACC // NTL
CH = 128
NCHUNK = 400
EPT = NCHUNK * CH
EEP = NTL * EPT

SB = 20
NSB = NCHUNK // SB


def _agg_body(
    table, src3, dst3, zrows, out,
    srcsb0, srcsb1, dstsb0, dstsb1, msgs_a, msgs_b, accum,
    semis0, semis1, semid0, semid1, sem_a, sem_b, sem_sa, sem_sb,
):
    c = lax.axis_index("c")
    s = lax.axis_index("s")
    lo = c * HALF

    def idx_issue(u, ssb, dsb, semis, semid):
        pltpu.async_copy(src3.at[s, pl.ds(u * SB, SB)], ssb, semis)
        pltpu.async_copy(dst3.at[s, pl.ds(u * SB, SB)], dsb, semid)

    idx_issue(0, srcsb0, dstsb0, semis0, semid0)
    idx_issue(1, srcsb1, dstsb1, semis1, semid1)

    r0 = s * ZPT
    pltpu.sync_copy(zrows.at[pl.ds(r0, ZPT)], accum.at[pl.ds(r0, ZPT)])
    plsc.subcore_barrier()

    def super_step(u, ssb, dsb, semis, semid):
        pltpu.make_async_copy(src3.at[s, pl.ds(0, SB)], ssb, semis).wait()

        def gissue(k, buf, sem):
            pltpu.async_copy(table.at[ssb.at[k]], buf, sem)

        def gwait(buf, sem):
            pltpu.make_async_copy(table.at[ssb.at[0]], buf, sem).wait()

        def sissue(k, buf, sem):
            pltpu.async_copy(buf, accum.at[dsb.at[k]], sem, add=True)

        def swait(buf, sem):
            pltpu.make_async_copy(buf, accum.at[dsb.at[0]], sem).wait()

        gissue(0, msgs_a, sem_a)
        gissue(1, msgs_b, sem_b)
        pltpu.make_async_copy(dst3.at[s, pl.ds(0, SB)], dsb, semid).wait()

        def remap(j, carry):
            for k in range(CH // 16):
                d = dsb[j, pl.ds(k * 16, 16)]
                loc = d - lo
                okm = (loc >= 0) & (loc < HALF)
                dsb[j, pl.ds(k * 16, 16)] = jnp.where(okm, loc, HALF)
            return carry

        lax.fori_loop(0, SB, remap, 0)

        def pair(v, carry):
            k0 = 2 * v
            gwait(msgs_a, sem_a)
            sissue(k0, msgs_a, sem_sa)
            gwait(msgs_b, sem_b)
            sissue(k0 + 1, msgs_b, sem_sb)
            swait(msgs_a, sem_sa)
            gissue(k0 + 2, msgs_a, sem_a)
            swait(msgs_b, sem_sb)
            gissue(k0 + 3, msgs_b, sem_b)
            return carry

        lax.fori_loop(0, (SB - 2) // 2, pair, 0)
        gwait(msgs_a, sem_a)
        sissue(SB - 2, msgs_a, sem_sa)
        gwait(msgs_b, sem_b)
        sissue(SB - 1, msgs_b, sem_sb)
        swait(msgs_a, sem_sa)
        swait(msgs_b, sem_sb)

        @pl.when(u + 2 < NSB)
        def _pref():
            idx_issue(u + 2, ssb, dsb, semis, semid)

    def outer(w, carry):
        super_step(2 * w, srcsb0, dstsb0, semis0, semid0)
        super_step(2 * w + 1, srcsb1, dstsb1, semis1, semid1)
        return carry

    lax.fori_loop(0, NSB // 2, outer, 0)
    plsc.subcore_barrier()

    start = jnp.minimum(s * ZPT, HALF - ZPT)
    pltpu.sync_copy(
        accum.at[pl.ds(start, ZPT)], out.at[pl.ds(c * HALF + start, ZPT)]
    )

    @pl.when(jnp.logical_and(c == NSC - 1, s == NTL - 1))
    def _pad_tail():
        pltpu.sync_copy(
            accum.at[pl.ds(HALF, NPAD - NN)], out.at[pl.ds(NN, NPAD - NN)]
        )


_agg_cache = []


def _agg(*args):
    if not _agg_cache:
        mesh = plsc.VectorSubcoreMesh(
            core_axis_name="c", subcore_axis_name="s",
            num_cores=NSC, num_subcores=NTL,
        )
        _agg_cache.append(pl.kernel(
            _agg_body,
            out_type=jax.ShapeDtypeStruct((NPAD, DD), jnp.float32),
            mesh=mesh,
            scratch_types=[
                pltpu.VMEM((SB, CH), jnp.int32),
                pltpu.VMEM((SB, CH), jnp.int32),
                pltpu.VMEM((SB, CH), jnp.int32),
                pltpu.VMEM((SB, CH), jnp.int32),
                pltpu.VMEM((CH, DD), jnp.float32),
                pltpu.VMEM((CH, DD), jnp.float32),
                pltpu.VMEM_SHARED((ACC, DD), jnp.float32),
                pltpu.SemaphoreType.DMA,
                pltpu.SemaphoreType.DMA,
                pltpu.SemaphoreType.DMA,
                pltpu.SemaphoreType.DMA,
                pltpu.SemaphoreType.DMA,
                pltpu.SemaphoreType.DMA,
                pltpu.SemaphoreType.DMA,
                pltpu.SemaphoreType.DMA,
            ],
            compiler_params=pltpu.CompilerParams(
                use_tc_tiling_on_sc=False, needs_layout_passes=False
            ),
        ))
    return _agg_cache[0](*args)


ACCF = ACC * DD
ZPTF = ACCF // NTL
TRASH = HALF * DD
XW = 784


def _agg1_body(
    xtab, src3, dst3, onesv, zflat, out,
    xloc, onesb, srcsb0, srcsb1, dstsb0, dstsb1, accum,
    semx, semis0, semis1, semid0, semid1, sem_s,
):
    c = lax.axis_index("c")
    s = lax.axis_index("s")
    lo = c * HALF

    pltpu.async_copy(xtab, xloc, semx)
    pltpu.sync_copy(onesv, onesb)

    def idx_issue(u, ssb, dsb, semis, semid):
        pltpu.async_copy(src3.at[s, pl.ds(u * SB, SB)], ssb, semis)
        pltpu.async_copy(dst3.at[s, pl.ds(u * SB, SB)], dsb, semid)

    idx_issue(0, srcsb0, dstsb0, semis0, semid0)
    idx_issue(1, srcsb1, dstsb1, semis1, semid1)

    r0 = s * ZPTF
    pltpu.sync_copy(zflat.at[pl.ds(r0, ZPTF)], accum.at[pl.ds(r0, ZPTF)])
    pltpu.make_async_copy(xtab, xloc, semx).wait()
    plsc.subcore_barrier()

    def super_step(u, ssb, dsb, semis, semid):
        pltpu.make_async_copy(src3.at[s, pl.ds(0, SB)], ssb, semis).wait()
        pltpu.make_async_copy(dst3.at[s, pl.ds(0, SB)], dsb, semid).wait()

        def chunk(j, carry):
            for k in range(CH // 16):
                sv = ssb[j, pl.ds(k * 16, 16)]
                w = sv >> 2
                xw = plsc.load_gather(xloc, [w >> 4, w & 15])
                xv = (xw >> ((sv & 3) << 3)) & 63
                d = dsb[j, pl.ds(k * 16, 16)]
                loc = d - lo
                okm = (loc >= 0) & (loc < HALF)
                p = loc * DD + xv
                dsb[j, pl.ds(k * 16, 16)] = jnp.where(okm, p, TRASH)
            return carry

        lax.fori_loop(0, SB, chunk, 0)

        def sc_issue(j, carry):
            pltpu.async_copy(onesb, accum.at[dsb.at[j]], sem_s, add=True)
            return carry

        lax.fori_loop(0, SB, sc_issue, 0)

        def sc_wait(j, carry):
            pltpu.make_async_copy(onesb, accum.at[dsb.at[0]], sem_s).wait()
            return carry

        lax.fori_loop(0, SB, sc_wait, 0)

        @pl.when(u + 2 < NSB)
        def _pref():
            idx_issue(u + 2, ssb, dsb, semis, semid)

    def outer(w, carry):
        super_step(2 * w, srcsb0, dstsb0, semis0, semid0)
        super_step(2 * w + 1, srcsb1, dstsb1, semis1, semid1)
        return carry

    lax.fori_loop(0, NSB // 2, outer, 0)
    plsc.subcore_barrier()

    start = jnp.minimum(s * ZPT, HALF - ZPT) * DD
    pltpu.sync_copy(
        accum.at[pl.ds(start, ZPT * DD)],
        out.at[pl.ds(c * HALF * DD + start, ZPT * DD)],
    )

    @pl.when(jnp.logical_and(c == NSC - 1, s == NTL - 1))
    def _pad_tail():
        pltpu.sync_copy(
            accum.at[pl.ds(HALF * DD, (NPAD - NN) * DD)],
            out.at[pl.ds(NN * DD, (NPAD - NN) * DD)],
        )


_agg1_cache = []


def _agg1(*args):
    if not _agg1_cache:
        mesh = plsc.VectorSubcoreMesh(
            core_axis_name="c", subcore_axis_name="s",
            num_cores=NSC, num_subcores=NTL,
        )
        _agg1_cache.append(pl.kernel(
            _agg1_body,
            out_type=jax.ShapeDtypeStruct((NPAD * DD,), jnp.float32),
            mesh=mesh,
            scratch_types=[
                pltpu.VMEM((XW, 16), jnp.int32),
                pltpu.VMEM((CH,), jnp.float32),
                pltpu.VMEM((SB, CH), jnp.int32),
                pltpu.VMEM((SB, CH), jnp.int32),
                pltpu.VMEM((SB, CH), jnp.int32),
                pltpu.VMEM((SB, CH), jnp.int32),
                pltpu.VMEM_SHARED((ACCF,), jnp.float32),
                pltpu.SemaphoreType.DMA,
                pltpu.SemaphoreType.DMA,
                pltpu.SemaphoreType.DMA,
                pltpu.SemaphoreType.DMA,
                pltpu.SemaphoreType.DMA,
                pltpu.SemaphoreType.DMA,
            ],
            compiler_params=pltpu.CompilerParams(
                use_tc_tiling_on_sc=False, needs_layout_passes=False
            ),
        ))
    return _agg1_cache[0](*args)


def _mmt(a, b):
    return lax.dot_general(
        a, b, (((1,), (1,)), ((), ())), preferred_element_type=jnp.float32
    )


def _mm(a, b):
    return lax.dot_general(
        a, b, (((1,), (0,)), ((), ())), preferred_element_type=jnp.float32
    )


def _layer1_body(c1_ref, x_ref, emb_ref, wl_ref, wr_ref, bl_ref, h1_ref):
    c1 = c1_ref[...]
    cnt = jnp.maximum(jnp.sum(c1, axis=1, keepdims=True), 1.0)
    meanw = c1 / cnt
    emb = emb_ref[...]
    m1 = _mmt(emb, wl_ref[...])
    r1 = _mmt(emb, wr_ref[...])
    xb = jnp.reshape(x_ref[...], (BN,))
    oh = (xb[:, None] == lax.broadcasted_iota(jnp.int32, (BN, DD), 1)).astype(
        jnp.float32
    )
    t = _mm(meanw, m1) + _mm(oh, r1) + bl_ref[...]
    h1_ref[...] = jnp.maximum(t, 0.0)


_layer1 = pl.pallas_call(
    _layer1_body,
    grid=(NB,),
    in_specs=[
        pl.BlockSpec((BN, DD), lambda i: (i, 0)),
        pl.BlockSpec((1, 1, BN), lambda i: (i, 0, 0)),
        pl.BlockSpec((DD, DD), lambda i: (0, 0)),
        pl.BlockSpec((DD, DD), lambda i: (0, 0)),
        pl.BlockSpec((DD, DD), lambda i: (0, 0)),
        pl.BlockSpec((1, DD), lambda i: (0, 0)),
    ],
    out_specs=pl.BlockSpec((BN, DD), lambda i: (i, 0)),
    out_shape=jax.ShapeDtypeStruct((NPAD, DD), jnp.float32),
)


def _layer2_body(
    s2_ref, h1_ref, c1_ref, b_ref, wl_ref, wr_ref, bl_ref, wlin_ref, blin_ref,
    out_ref, p_scr,
):
    j = pl.program_id(0)

    @pl.when(j == 0)
    def _init():
        p_scr[...] = jnp.zeros_like(p_scr)

    c1 = c1_ref[...]
    cnt = jnp.maximum(jnp.sum(c1, axis=1, keepdims=True), 1.0)
    t = _mmt(s2_ref[...] / cnt, wl_ref[...]) + _mmt(h1_ref[...], wr_ref[...])
    h2 = jnp.maximum(t + bl_ref[...], 0.0)
    bb = jnp.reshape(b_ref[...], (BN,))
    ohg = (bb[:, None] == lax.broadcasted_iota(jnp.int32, (BN, GG), 1)).astype(
        jnp.float32
    )
    hw = jnp.concatenate([h2, jnp.zeros((BN, DD), jnp.float32)], axis=1)
    li = lax.broadcasted_iota(jnp.int32, (BN, 2 * DD), 1)
    hw = jnp.where(li == DD, 1.0, hw)
    p_scr[...] += lax.dot_general(
        ohg, hw, (((0,), (0,)), ((), ())), preferred_element_type=jnp.float32
    )

    @pl.when(j == NB - 1)
    def _final():
        p = p_scr[...]
        gc = jnp.maximum(p[:, DD : DD + 1], 1.0)
        pooled = p[:, :DD] / gc
        out_ref[...] = _mmt(pooled, wlin_ref[...]) + blin_ref[...]


_layer2 = pl.pallas_call(
    _layer2_body,
    grid=(NB,),
    in_specs=[
        pl.BlockSpec((BN, DD), lambda i: (i, 0)),
        pl.BlockSpec((BN, DD), lambda i: (i, 0)),
        pl.BlockSpec((BN, DD), lambda i: (i, 0)),
        pl.BlockSpec((1, 1, BN), lambda i: (i, 0, 0)),
        pl.BlockSpec((DD, DD), lambda i: (0, 0)),
        pl.BlockSpec((DD, DD), lambda i: (0, 0)),
        pl.BlockSpec((1, DD), lambda i: (0, 0)),
        pl.BlockSpec((CC, DD), lambda i: (0, 0)),
        pl.BlockSpec((1, CC), lambda i: (0, 0)),
    ],
    out_specs=pl.BlockSpec((GG, CC), lambda i: (0, 0)),
    out_shape=jax.ShapeDtypeStruct((GG, CC), jnp.float32),
    scratch_shapes=[pltpu.VMEM((GG, 2 * DD), jnp.float32)],
)


def kernel(x, edge_index, batch, emb, Wl1, bl1, Wr1, Wl2, bl2, Wr2, Wlin, blin):
    src = jnp.reshape(
        jnp.pad(edge_index[0].astype(jnp.int32), (0, EEP - EE)),
        (NTL, NCHUNK, CH),
    )
    dst = jnp.reshape(
        jnp.pad(
            edge_index[1].astype(jnp.int32), (0, EEP - EE),
            constant_values=-1,
        ),
        (NTL, NCHUNK, CH),
    )
    xp3 = jnp.reshape(
        jnp.pad(x.astype(jnp.int32), (0, NPAD - NN)), (NB, 1, BN)
    )
    b3 = jnp.reshape(
        jnp.pad(batch.astype(jnp.int32), (0, NPAD - NN), constant_values=2**20),
        (NB, 1, BN),
    )
    zrows = jnp.zeros((ACC, DD), jnp.float32)

    xpk = jnp.pad(x.astype(jnp.int32), (0, XW * 64 - NN))
    xpk = jnp.reshape(xpk, (XW * 16, 4))
    xi = jnp.reshape(
        xpk[:, 0] | (xpk[:, 1] << 8) | (xpk[:, 2] << 16) | (xpk[:, 3] << 24),
        (XW, 16),
    )
    c1 = jnp.reshape(
        _agg1(
            xi, src, dst, jnp.ones((CH,), jnp.float32),
            jnp.reshape(zrows, (ACCF,)),
        ),
        (NPAD, DD),
    )
    h1 = _layer1(c1, xp3, emb, Wl1, Wr1, jnp.reshape(bl1, (1, DD)))
    s2 = _agg(h1, src, dst, zrows)
    out = _layer2(
        s2, h1, c1, b3, Wl2, Wr2,
        jnp.reshape(bl2, (1, DD)), Wlin, jnp.reshape(blin, (1, CC)),
    )
    return out


# speedup vs baseline: 1.4235x; 1.4235x over previous
import jax
import jax.numpy as jnp
from jax import lax
from jax.experimental import pallas as pl
from jax.experimental.pallas import tpu as pltpu
from jax.experimental.pallas import tpu_sc as plsc

NN = 50000
EE = 800000
DD = 64
GG = 1024
CC = 10

BN = 256
NB = 196
NPAD = NB * BN

NSC = 2
NTL = 16
HALF = NN // NSC
ACC = 25600
ZPT = ACC // NTL
CH = 80
NCHUNK = 625
EPT = NCHUNK * CH
EEP = NTL * EPT

SB = 25
NSB = NCHUNK // SB


def _agg_body(
    table, src3, dst3, zrows, out,
    srcsb0, srcsb1, dstsb0, dstsb1, msgs_a, msgs_b, accum,
    semis0, semis1, semid0, semid1, sem_a, sem_b, sem_sa, sem_sb,
):
    c = lax.axis_index("c")
    s = lax.axis_index("s")
    lo = c * HALF

    def idx_issue(u, ssb, dsb, semis, semid):
        pltpu.async_copy(src3.at[s, pl.ds(u * SB, SB)], ssb, semis)
        pltpu.async_copy(dst3.at[s, pl.ds(u * SB, SB)], dsb, semid)

    idx_issue(0, srcsb0, dstsb0, semis0, semid0)
    idx_issue(1, srcsb1, dstsb1, semis1, semid1)

    r0 = s * ZPT
    pltpu.sync_copy(zrows.at[pl.ds(r0, ZPT)], accum.at[pl.ds(r0, ZPT)])
    plsc.subcore_barrier()

    def super_step(u, ssb, dsb, semis, semid):
        pltpu.make_async_copy(src3.at[s, pl.ds(0, SB)], ssb, semis).wait()

        def gissue(k, buf, sem):
            pltpu.async_copy(table.at[ssb.at[k]], buf, sem)

        def gwait(buf, sem):
            pltpu.make_async_copy(table.at[ssb.at[0]], buf, sem).wait()

        def sissue(k, buf, sem):
            pltpu.async_copy(buf, accum.at[dsb.at[k]], sem, add=True)

        def swait(buf, sem):
            pltpu.make_async_copy(buf, accum.at[dsb.at[0]], sem).wait()

        gissue(0, msgs_a, sem_a)
        gissue(1, msgs_b, sem_b)
        pltpu.make_async_copy(dst3.at[s, pl.ds(0, SB)], dsb, semid).wait()

        def remap(j, carry):
            for k in range(CH // 16):
                d = dsb[j, pl.ds(k * 16, 16)]
                loc = d - lo
                okm = (loc >= 0) & (loc < HALF)
                dsb[j, pl.ds(k * 16, 16)] = jnp.where(okm, loc, HALF)
            return carry

        lax.fori_loop(0, SB, remap, 0)

        def pair(v, carry):
            k0 = 2 * v
            gwait(msgs_a, sem_a)
            sissue(k0, msgs_a, sem_sa)
            gwait(msgs_b, sem_b)
            sissue(k0 + 1, msgs_b, sem_sb)
            swait(msgs_a, sem_sa)
            gissue(k0 + 2, msgs_a, sem_a)
            swait(msgs_b, sem_sb)
            gissue(k0 + 3, msgs_b, sem_b)
            return carry

        lax.fori_loop(0, (SB - 3) // 2, pair, 0)
        gwait(msgs_a, sem_a)
        sissue(SB - 3, msgs_a, sem_sa)
        gwait(msgs_b, sem_b)
        sissue(SB - 2, msgs_b, sem_sb)
        swait(msgs_a, sem_sa)
        gissue(SB - 1, msgs_a, sem_a)
        gwait(msgs_a, sem_a)
        sissue(SB - 1, msgs_a, sem_sa)
        swait(msgs_b, sem_sb)
        swait(msgs_a, sem_sa)

        @pl.when(u + 2 < NSB)
        def _pref():
            idx_issue(u + 2, ssb, dsb, semis, semid)

    def outer(w, carry):
        super_step(2 * w, srcsb0, dstsb0, semis0, semid0)
        super_step(2 * w + 1, srcsb1, dstsb1, semis1, semid1)
        return carry

    lax.fori_loop(0, (NSB - 1) // 2, outer, 0)
    super_step(NSB - 1, srcsb0, dstsb0, semis0, semid0)
    plsc.subcore_barrier()

    start = jnp.minimum(s * ZPT, HALF - ZPT)
    pltpu.sync_copy(
        accum.at[pl.ds(start, ZPT)], out.at[pl.ds(c * HALF + start, ZPT)]
    )

    @pl.when(jnp.logical_and(c == NSC - 1, s == NTL - 1))
    def _pad_tail():
        pltpu.sync_copy(
            accum.at[pl.ds(HALF, NPAD - NN)], out.at[pl.ds(NN, NPAD - NN)]
        )


_agg_cache = []


def _agg(*args):
    if not _agg_cache:
        mesh = plsc.VectorSubcoreMesh(
            core_axis_name="c", subcore_axis_name="s",
            num_cores=NSC, num_subcores=NTL,
        )
        _agg_cache.append(pl.kernel(
            _agg_body,
            out_type=jax.ShapeDtypeStruct((NPAD, DD), jnp.float32),
            mesh=mesh,
            scratch_types=[
                pltpu.VMEM((SB, CH), jnp.int32),
                pltpu.VMEM((SB, CH), jnp.int32),
                pltpu.VMEM((SB, CH), jnp.int32),
                pltpu.VMEM((SB, CH), jnp.int32),
                pltpu.VMEM((CH, DD), jnp.float32),
                pltpu.VMEM((CH, DD), jnp.float32),
                pltpu.VMEM_SHARED((ACC, DD), jnp.float32),
                pltpu.SemaphoreType.DMA,
                pltpu.SemaphoreType.DMA,
                pltpu.SemaphoreType.DMA,
                pltpu.SemaphoreType.DMA,
                pltpu.SemaphoreType.DMA,
                pltpu.SemaphoreType.DMA,
                pltpu.SemaphoreType.DMA,
                pltpu.SemaphoreType.DMA,
            ],
            compiler_params=pltpu.CompilerParams(
                use_tc_tiling_on_sc=False, needs_layout_passes=False
            ),
        ))
    return _agg_cache[0](*args)


ACCF = ACC * DD
ZPTF = ACCF // NTL
TRASH = HALF * DD
XW = 784


def _agg1_body(
    xtab, src3, dst3, onesv, zflat, out,
    xloc, onesb, srcsb0, srcsb1, dstsb0, dstsb1, accum,
    semx, semis0, semis1, semid0, semid1, sem_s,
):
    c = lax.axis_index("c")
    s = lax.axis_index("s")
    lo = c * HALF

    pltpu.async_copy(xtab, xloc, semx)
    pltpu.sync_copy(onesv, onesb)

    def idx_issue(u, ssb, dsb, semis, semid):
        pltpu.async_copy(src3.at[s, pl.ds(u * SB, SB)], ssb, semis)
        pltpu.async_copy(dst3.at[s, pl.ds(u * SB, SB)], dsb, semid)

    idx_issue(0, srcsb0, dstsb0, semis0, semid0)
    idx_issue(1, srcsb1, dstsb1, semis1, semid1)

    r0 = s * ZPTF
    pltpu.sync_copy(zflat.at[pl.ds(r0, ZPTF)], accum.at[pl.ds(r0, ZPTF)])
    pltpu.make_async_copy(xtab, xloc, semx).wait()
    plsc.subcore_barrier()

    def super_step(u, ssb, dsb, semis, semid):
        pltpu.make_async_copy(src3.at[s, pl.ds(0, SB)], ssb, semis).wait()
        pltpu.make_async_copy(dst3.at[s, pl.ds(0, SB)], dsb, semid).wait()

        def chunk(j, carry):
            for k in range(CH // 16):
                sv = ssb[j, pl.ds(k * 16, 16)]
                w = sv >> 2
                xw = plsc.load_gather(xloc, [w >> 4, w & 15])
                xv = (xw >> ((sv & 3) << 3)) & 63
                d = dsb[j, pl.ds(k * 16, 16)]
                loc = d - lo
                okm = (loc >= 0) & (loc < HALF)
                p = loc * DD + xv
                dsb[j, pl.ds(k * 16, 16)] = jnp.where(okm, p, TRASH)
            pltpu.async_copy(onesb, accum.at[dsb.at[j]], sem_s, add=True)
            return carry

        lax.fori_loop(0, SB, chunk, 0)

        def sc_wait(j, carry):
            pltpu.make_async_copy(onesb, accum.at[dsb.at[0]], sem_s).wait()
            return carry

        lax.fori_loop(0, SB, sc_wait, 0)

        @pl.when(u + 2 < NSB)
        def _pref():
            idx_issue(u + 2, ssb, dsb, semis, semid)

    def outer(w, carry):
        super_step(2 * w, srcsb0, dstsb0, semis0, semid0)
        super_step(2 * w + 1, srcsb1, dstsb1, semis1, semid1)
        return carry

    lax.fori_loop(0, (NSB - 1) // 2, outer, 0)
    super_step(NSB - 1, srcsb0, dstsb0, semis0, semid0)
    plsc.subcore_barrier()

    start = jnp.minimum(s * ZPT, HALF - ZPT) * DD
    pltpu.sync_copy(
        accum.at[pl.ds(start, ZPT * DD)],
        out.at[pl.ds(c * HALF * DD + start, ZPT * DD)],
    )

    @pl.when(jnp.logical_and(c == NSC - 1, s == NTL - 1))
    def _pad_tail():
        pltpu.sync_copy(
            accum.at[pl.ds(HALF * DD, (NPAD - NN) * DD)],
            out.at[pl.ds(NN * DD, (NPAD - NN) * DD)],
        )


_agg1_cache = []


def _agg1(*args):
    if not _agg1_cache:
        mesh = plsc.VectorSubcoreMesh(
            core_axis_name="c", subcore_axis_name="s",
            num_cores=NSC, num_subcores=NTL,
        )
        _agg1_cache.append(pl.kernel(
            _agg1_body,
            out_type=jax.ShapeDtypeStruct((NPAD * DD,), jnp.float32),
            mesh=mesh,
            scratch_types=[
                pltpu.VMEM((XW, 16), jnp.int32),
                pltpu.VMEM((CH,), jnp.float32),
                pltpu.VMEM((SB, CH), jnp.int32),
                pltpu.VMEM((SB, CH), jnp.int32),
                pltpu.VMEM((SB, CH), jnp.int32),
                pltpu.VMEM((SB, CH), jnp.int32),
                pltpu.VMEM_SHARED((ACCF,), jnp.float32),
                pltpu.SemaphoreType.DMA,
                pltpu.SemaphoreType.DMA,
                pltpu.SemaphoreType.DMA,
                pltpu.SemaphoreType.DMA,
                pltpu.SemaphoreType.DMA,
                pltpu.SemaphoreType.DMA,
            ],
            compiler_params=pltpu.CompilerParams(
                use_tc_tiling_on_sc=False, needs_layout_passes=False
            ),
        ))
    return _agg1_cache[0](*args)


def _mmt(a, b):
    return lax.dot_general(
        a, b, (((1,), (1,)), ((), ())), preferred_element_type=jnp.float32
    )


def _mm(a, b):
    return lax.dot_general(
        a, b, (((1,), (0,)), ((), ())), preferred_element_type=jnp.float32
    )


def _layer1_body(c1_ref, x_ref, emb_ref, wl_ref, wr_ref, bl_ref, h1_ref):
    c1 = c1_ref[...]
    cnt = jnp.maximum(jnp.sum(c1, axis=1, keepdims=True), 1.0)
    meanw = c1 / cnt
    emb = emb_ref[...]
    m1 = _mmt(emb, wl_ref[...])
    r1 = _mmt(emb, wr_ref[...])
    xb = jnp.reshape(x_ref[...], (BN,))
    oh = (xb[:, None] == lax.broadcasted_iota(jnp.int32, (BN, DD), 1)).astype(
        jnp.float32
    )
    t = _mm(meanw, m1) + _mm(oh, r1) + bl_ref[...]
    h1_ref[...] = jnp.maximum(t, 0.0)


_layer1 = pl.pallas_call(
    _layer1_body,
    grid=(NB,),
    in_specs=[
        pl.BlockSpec((BN, DD), lambda i: (i, 0)),
        pl.BlockSpec((1, 1, BN), lambda i: (i, 0, 0)),
        pl.BlockSpec((DD, DD), lambda i: (0, 0)),
        pl.BlockSpec((DD, DD), lambda i: (0, 0)),
        pl.BlockSpec((DD, DD), lambda i: (0, 0)),
        pl.BlockSpec((1, DD), lambda i: (0, 0)),
    ],
    out_specs=pl.BlockSpec((BN, DD), lambda i: (i, 0)),
    out_shape=jax.ShapeDtypeStruct((NPAD, DD), jnp.float32),
)


def _layer2_body(
    s2_ref, h1_ref, c1_ref, b_ref, wl_ref, wr_ref, bl_ref, wlin_ref, blin_ref,
    out_ref, p_scr,
):
    j = pl.program_id(0)

    @pl.when(j == 0)
    def _init():
        p_scr[...] = jnp.zeros_like(p_scr)

    c1 = c1_ref[...]
    cnt = jnp.maximum(jnp.sum(c1, axis=1, keepdims=True), 1.0)
    t = _mmt(s2_ref[...] / cnt, wl_ref[...]) + _mmt(h1_ref[...], wr_ref[...])
    h2 = jnp.maximum(t + bl_ref[...], 0.0)
    bb = jnp.reshape(b_ref[...], (BN,))
    ohg = (bb[:, None] == lax.broadcasted_iota(jnp.int32, (BN, GG), 1)).astype(
        jnp.float32
    )
    hw = jnp.concatenate([h2, jnp.zeros((BN, DD), jnp.float32)], axis=1)
    li = lax.broadcasted_iota(jnp.int32, (BN, 2 * DD), 1)
    hw = jnp.where(li == DD, 1.0, hw)
    p_scr[...] += lax.dot_general(
        ohg, hw, (((0,), (0,)), ((), ())), preferred_element_type=jnp.float32
    )

    @pl.when(j == NB - 1)
    def _final():
        p = p_scr[...]
        gc = jnp.maximum(p[:, DD : DD + 1], 1.0)
        pooled = p[:, :DD] / gc
        out_ref[...] = _mmt(pooled, wlin_ref[...]) + blin_ref[...]


_layer2 = pl.pallas_call(
    _layer2_body,
    grid=(NB,),
    in_specs=[
        pl.BlockSpec((BN, DD), lambda i: (i, 0)),
        pl.BlockSpec((BN, DD), lambda i: (i, 0)),
        pl.BlockSpec((BN, DD), lambda i: (i, 0)),
        pl.BlockSpec((1, 1, BN), lambda i: (i, 0, 0)),
        pl.BlockSpec((DD, DD), lambda i: (0, 0)),
        pl.BlockSpec((DD, DD), lambda i: (0, 0)),
        pl.BlockSpec((1, DD), lambda i: (0, 0)),
        pl.BlockSpec((CC, DD), lambda i: (0, 0)),
        pl.BlockSpec((1, CC), lambda i: (0, 0)),
    ],
    out_specs=pl.BlockSpec((GG, CC), lambda i: (0, 0)),
    out_shape=jax.ShapeDtypeStruct((GG, CC), jnp.float32),
    scratch_shapes=[pltpu.VMEM((GG, 2 * DD), jnp.float32)],
)


def kernel(x, edge_index, batch, emb, Wl1, bl1, Wr1, Wl2, bl2, Wr2, Wlin, blin):
    src = jnp.reshape(
        jnp.pad(edge_index[0].astype(jnp.int32), (0, EEP - EE)),
        (NTL, NCHUNK, CH),
    )
    dst = jnp.reshape(
        jnp.pad(
            edge_index[1].astype(jnp.int32), (0, EEP - EE),
            constant_values=-1,
        ),
        (NTL, NCHUNK, CH),
    )
    xp3 = jnp.reshape(
        jnp.pad(x.astype(jnp.int32), (0, NPAD - NN)), (NB, 1, BN)
    )
    b3 = jnp.reshape(
        jnp.pad(batch.astype(jnp.int32), (0, NPAD - NN), constant_values=2**20),
        (NB, 1, BN),
    )
    zrows = jnp.zeros((ACC, DD), jnp.float32)

    xpk = jnp.pad(x.astype(jnp.int32), (0, XW * 64 - NN))
    xpk = jnp.reshape(xpk, (XW * 16, 4))
    xi = jnp.reshape(
        xpk[:, 0] | (xpk[:, 1] << 8) | (xpk[:, 2] << 16) | (xpk[:, 3] << 24),
        (XW, 16),
    )
    c1 = jnp.reshape(
        _agg1(
            xi, src, dst, jnp.ones((CH,), jnp.float32),
            jnp.reshape(zrows, (ACCF,)),
        ),
        (NPAD, DD),
    )
    h1 = _layer1(c1, xp3, emb, Wl1, Wr1, jnp.reshape(bl1, (1, DD)))
    s2 = _agg(h1, src, dst, zrows)
    out = _layer2(
        s2, h1, c1, b3, Wl2, Wr2,
        jnp.reshape(bl2, (1, DD)), Wlin, jnp.reshape(blin, (1, CC)),
    )
    return out
